# Initial kernel scaffold; baseline (speedup 1.0000x reference)
import jax
import jax.numpy as jnp
from jax.experimental import pallas as pl


def kernel(X_train, y_train, X_missing):
    raise NotImplementedError("write your pallas kernel here")


# speedup vs baseline: 8.3909x; 8.3909x over previous
import jax
import jax.numpy as jnp
from jax.experimental import pallas as pl
from jax.experimental.pallas import tpu as pltpu

_K = 16
_TAU = 1.0


def _dknn_body(x_ref, q_ref, y_ref, o_ref):
    x = x_ref[...]
    q = q_ref[...]
    y = y_ref[...]
    qn = q.shape[0]

    x2 = jnp.sum(x * x, axis=1, keepdims=True)
    q2 = jnp.sum(q * q, axis=1)[None, :]
    s = jax.lax.dot_general(
        x, q, (((1,), (1,)), ((), ())),
        preferred_element_type=jnp.float32,
        precision=jax.lax.Precision.DEFAULT)
    d2 = x2 + q2 - 2.0 * s
    neg = -jnp.sqrt(jnp.maximum(d2, 1e-12)) * (1.0 / _TAU)

    m0 = jnp.max(neg, axis=1, keepdims=True)
    e = jnp.exp(neg - m0)
    z = jnp.sum(e, axis=1, keepdims=True)

    iota = jax.lax.broadcasted_iota(jnp.int32, e.shape, 1)
    work = e
    mask = jnp.zeros(e.shape, dtype=jnp.bool_)
    for _ in range(_K):
        m = jnp.max(work, axis=1, keepdims=True)
        cand = jnp.where(work == m, iota, qn)
        j = jnp.min(cand, axis=1, keepdims=True)
        sel = iota == j
        mask = jnp.logical_or(mask, sel)
        work = jnp.where(sel, -jnp.inf, work)

    em = jnp.where(mask, e, 0.0)
    out = jax.lax.dot_general(
        em, y, (((1,), (0,)), ((), ())),
        preferred_element_type=jnp.float32,
        precision=jax.lax.Precision.HIGHEST)
    o_ref[...] = out / z


def kernel(X_train, y_train, X_missing):
    n, d = X_train.shape
    qn = X_missing.shape[0]
    l = y_train.shape[1]
    y_q = y_train[:qn]

    tn = 2000
    assert n % tn == 0
    out = pl.pallas_call(
        _dknn_body,
        grid=(n // tn,),
        in_specs=[
            pl.BlockSpec((tn, d), lambda i: (i, 0)),
            pl.BlockSpec((qn, d), lambda i: (0, 0)),
            pl.BlockSpec((qn, l), lambda i: (0, 0)),
        ],
        out_specs=pl.BlockSpec((tn, l), lambda i: (i, 0)),
        out_shape=jax.ShapeDtypeStruct((n, l), jnp.float32),
        compiler_params=pltpu.CompilerParams(
            dimension_semantics=("parallel",)),
    )(X_train, X_missing, y_q)
    return out[None]


# speedup vs baseline: 22.6202x; 2.6958x over previous
import jax
import jax.numpy as jnp
from jax.experimental import pallas as pl
from jax.experimental.pallas import tpu as pltpu

_K = 16
_TAU = 1.0
_TN = 2048


def _dknn_body(x_ref, q_ref, y_ref, o_ref):
    x = x_ref[...]
    q = q_ref[...]
    y = y_ref[...]

    x2 = jnp.sum(x * x, axis=1)[None, :]
    q2 = jnp.sum(q * q, axis=1)[:, None]
    s = jax.lax.dot_general(
        q, x, (((1,), (1,)), ((), ())),
        preferred_element_type=jnp.float32,
        precision=jax.lax.Precision.DEFAULT)
    d2 = q2 + x2 - 2.0 * s
    neg = -jnp.sqrt(jnp.maximum(d2, 1e-12)) * (1.0 / _TAU)

    m0 = jnp.max(neg, axis=0, keepdims=True)
    e = jnp.exp(neg - m0)
    z = jnp.sum(e, axis=0, keepdims=True)

    work = jnp.where(e >= 1.0, -jnp.inf, e)
    for _ in range(_K - 2):
        m = jnp.max(work, axis=0, keepdims=True)
        work = jnp.where(work >= m, -jnp.inf, work)
    t = jnp.max(work, axis=0, keepdims=True)
    em = jnp.where(e >= t, e, 0.0)

    out_t = jax.lax.dot_general(
        y, em, (((0,), (0,)), ((), ())),
        preferred_element_type=jnp.float32,
        precision=jax.lax.Precision.HIGHEST)
    o_ref[...] = out_t / z


def kernel(X_train, y_train, X_missing):
    n, d = X_train.shape
    qn = X_missing.shape[0]
    l = y_train.shape[1]
    y_q = y_train[:qn]

    out_t = pl.pallas_call(
        _dknn_body,
        grid=(pl.cdiv(n, _TN),),
        in_specs=[
            pl.BlockSpec((_TN, d), lambda i: (i, 0)),
            pl.BlockSpec((qn, d), lambda i: (0, 0)),
            pl.BlockSpec((qn, l), lambda i: (0, 0)),
        ],
        out_specs=pl.BlockSpec((l, _TN), lambda i: (0, i)),
        out_shape=jax.ShapeDtypeStruct((l, n), jnp.float32),
        compiler_params=pltpu.CompilerParams(
            dimension_semantics=("parallel",)),
    )(X_train, X_missing, y_q)
    return out_t.T[None]


# speedup vs baseline: 27.7808x; 1.2281x over previous
import jax
import jax.numpy as jnp
from jax.experimental import pallas as pl
from jax.experimental.pallas import tpu as pltpu

_K = 16
_TAU = 1.0
_TN = 2048


def _dknn_body(x_ref, q_ref, y_ref, o_ref):
    x = x_ref[...]
    q = q_ref[...]
    y = y_ref[...]

    x2 = jnp.sum(x * x, axis=1)[None, :]
    q2 = jnp.sum(q * q, axis=1)[:, None]
    s = jax.lax.dot_general(
        q, x, (((1,), (1,)), ((), ())),
        preferred_element_type=jnp.float32,
        precision=jax.lax.Precision.DEFAULT)
    d2 = q2 + x2 - 2.0 * s

    _BIG = jnp.float32(3.4e38)
    m1 = jnp.min(d2, axis=0, keepdims=True)
    work = jnp.where(d2 <= m1, _BIG, d2)
    for _ in range(_K - 2):
        m = jnp.min(work, axis=0, keepdims=True)
        work = jnp.where(work <= m, _BIG, work)
    t2 = jnp.min(work, axis=0, keepdims=True)

    d = jnp.sqrt(jnp.maximum(d2, 1e-12))
    e = jnp.exp((jnp.sqrt(jnp.maximum(m1, 1e-12)) - d) * (1.0 / _TAU))
    z = jnp.sum(e, axis=0, keepdims=True)
    em = jnp.where(d2 <= t2, e, 0.0)

    out_t = jax.lax.dot_general(
        y, em, (((0,), (0,)), ((), ())),
        preferred_element_type=jnp.float32,
        precision=jax.lax.Precision.DEFAULT)
    o_ref[...] = out_t / z


def kernel(X_train, y_train, X_missing):
    n, d = X_train.shape
    qn = X_missing.shape[0]
    l = y_train.shape[1]
    y_q = y_train[:qn]

    out_t = pl.pallas_call(
        _dknn_body,
        grid=(pl.cdiv(n, _TN),),
        in_specs=[
            pl.BlockSpec((_TN, d), lambda i: (i, 0)),
            pl.BlockSpec((qn, d), lambda i: (0, 0)),
            pl.BlockSpec((qn, l), lambda i: (0, 0)),
        ],
        out_specs=pl.BlockSpec((l, _TN), lambda i: (0, i)),
        out_shape=jax.ShapeDtypeStruct((l, n), jnp.float32),
        compiler_params=pltpu.CompilerParams(
            dimension_semantics=("parallel",)),
    )(X_train, X_missing, y_q)
    return out_t.T[None]
